# baseline (speedup 1.0000x reference)
import jax
import jax.numpy as jnp
from jax import lax
from jax.experimental import pallas as pl
from jax.experimental.pallas import tpu as pltpu
from jax.experimental.pallas import tpu_sc as plsc

N_NODES = 10000
N_EDGES = 320000
D_FEAT = 128
DEG_W = 16
NC, NS = 2, 16
NW = NC * NS
CHUNK = 128
N_CHUNKS = N_EDGES // CHUNK
CH_PER_TILE = N_CHUNKS // NW
N_TAIL = N_CHUNKS - CH_PER_TILE * NW
SEC_CHUNKS = 6
N_SEC = CH_PER_TILE // SEC_CHUNKS
ROWS_PER_TILE = N_NODES // NS


def _sc_body(x_hbm, ei_hbm, zeros_hbm, zerod_hbm, ones_hbm,
             out_hbm, deg_hbm,
             isec0, isec1, rows0_v, rows1_v, ones_v,
             acc_sh, deg_sh, isem0, isem1, gsem0, gsem1, osem0, osem1):
    c = lax.axis_index("c")
    s = lax.axis_index("s")
    wid = c * NS + s
    base = wid * CH_PER_TILE

    isecs, isems = (isec0, isec1), (isem0, isem1)
    bufs, gsems = (rows0_v, rows1_v), (gsem0, gsem1)
    osems = (osem0, osem1)

    pltpu.async_copy(ei_hbm.at[pl.ds(base, SEC_CHUNKS)], isec0, isem0)
    pltpu.sync_copy(ones_hbm, ones_v)
    row0 = pl.multiple_of(s * ROWS_PER_TILE, 8)
    pltpu.sync_copy(zeros_hbm, acc_sh.at[pl.ds(row0, ROWS_PER_TILE)])
    pltpu.sync_copy(zerod_hbm, deg_sh.at[pl.ds(row0, ROWS_PER_TILE)])
    plsc.subcore_barrier()

    pltpu.make_async_copy(ei_hbm.at[pl.ds(base, SEC_CHUNKS)],
                          isec0, isem0).wait()
    pltpu.async_copy(x_hbm.at[isec0.at[0, 0]], rows0_v, gsem0)
    pltpu.async_copy(x_hbm.at[isec0.at[1, 0]], rows1_v, gsem1)

    for sec in range(N_SEC):
        ib = isecs[sec % 2]
        nb = (sec + 1) % 2
        if sec + 1 < N_SEC:
            pltpu.async_copy(
                ei_hbm.at[pl.ds(base + (sec + 1) * SEC_CHUNKS, SEC_CHUNKS)],
                isecs[nb], isems[nb])

        pltpu.async_copy(ones_v, deg_sh.at[ib.at[0, 1]], osem0, add=True)
        pltpu.async_copy(ones_v, deg_sh.at[ib.at[1, 1]], osem1, add=True)

        def body(j, carry):
            for b in (0, 1):
                k = 2 * j + b
                pltpu.make_async_copy(x_hbm.at[ib.at[k, 0]],
                                      bufs[b], gsems[b]).wait()
                pltpu.sync_copy(bufs[b], acc_sh.at[ib.at[k, 1]], add=True)
                pltpu.async_copy(x_hbm.at[ib.at[k + 2, 0]], bufs[b], gsems[b])
                pltpu.make_async_copy(ones_v, deg_sh.at[ib.at[k, 1]],
                                      osems[b]).wait()
                pltpu.async_copy(ones_v, deg_sh.at[ib.at[k + 2, 1]],
                                 osems[b], add=True)
            return carry

        lax.fori_loop(0, SEC_CHUNKS // 2 - 1, body, 0)

        for b in (0, 1):
            k = SEC_CHUNKS - 2 + b
            pltpu.make_async_copy(x_hbm.at[ib.at[k, 0]],
                                  bufs[b], gsems[b]).wait()
            pltpu.sync_copy(bufs[b], acc_sh.at[ib.at[k, 1]], add=True)
            if sec + 1 < N_SEC:
                if b == 0:
                    pltpu.make_async_copy(
                        ei_hbm.at[pl.ds(base + (sec + 1) * SEC_CHUNKS,
                                        SEC_CHUNKS)],
                        isecs[nb], isems[nb]).wait()
                pltpu.async_copy(x_hbm.at[isecs[nb].at[b, 0]],
                                 bufs[b], gsems[b])
            pltpu.make_async_copy(ones_v, deg_sh.at[ib.at[k, 1]],
                                  osems[b]).wait()

    pltpu.sync_copy(ei_hbm.at[pl.ds(NW * CH_PER_TILE, N_TAIL)],
                    isec0.at[pl.ds(0, N_TAIL)])
    cxl = wid // 8
    off = pl.multiple_of((wid % 8) * 16, 8)
    pltpu.async_copy(x_hbm.at[isec0.at[cxl, 0, pl.ds(off, 16)]],
                     rows0_v.at[pl.ds(0, 16)], gsem0).wait()
    pltpu.sync_copy(rows0_v.at[pl.ds(0, 16)],
                    acc_sh.at[isec0.at[cxl, 1, pl.ds(off, 16)]], add=True)
    pltpu.sync_copy(ones_v.at[pl.ds(0, 16)],
                    deg_sh.at[isec0.at[cxl, 1, pl.ds(off, 16)]], add=True)

    plsc.subcore_barrier()

    orow = pl.multiple_of(c * N_NODES + s * ROWS_PER_TILE, 8)
    pltpu.sync_copy(acc_sh.at[pl.ds(row0, ROWS_PER_TILE)],
                    out_hbm.at[pl.ds(orow, ROWS_PER_TILE)])
    ocol = pl.multiple_of(c * DEG_W, 8)
    pltpu.sync_copy(deg_sh.at[pl.ds(row0, ROWS_PER_TILE)],
                    deg_hbm.at[pl.ds(row0, ROWS_PER_TILE), pl.ds(ocol, DEG_W)])


_sc_call = pl.kernel(
    _sc_body,
    out_type=(
        jax.ShapeDtypeStruct((2 * N_NODES, D_FEAT), jnp.float32),
        jax.ShapeDtypeStruct((N_NODES, D_FEAT), jnp.float32),
    ),
    mesh=plsc.VectorSubcoreMesh(core_axis_name="c", subcore_axis_name="s"),
    compiler_params=pltpu.CompilerParams(use_tc_tiling_on_sc=False),
    scratch_types=(
        pltpu.VMEM((SEC_CHUNKS, 2, CHUNK), jnp.int32),
        pltpu.VMEM((SEC_CHUNKS, 2, CHUNK), jnp.int32),
        pltpu.VMEM((CHUNK, D_FEAT), jnp.float32),
        pltpu.VMEM((CHUNK, D_FEAT), jnp.float32),
        pltpu.VMEM((CHUNK, DEG_W), jnp.float32),
        pltpu.VMEM_SHARED((N_NODES, D_FEAT), jnp.float32),
        pltpu.VMEM_SHARED((N_NODES, DEG_W), jnp.float32),
        pltpu.SemaphoreType.DMA,
        pltpu.SemaphoreType.DMA,
        pltpu.SemaphoreType.DMA,
        pltpu.SemaphoreType.DMA,
        pltpu.SemaphoreType.DMA,
        pltpu.SemaphoreType.DMA,
    ),
)


_BLK = 2000


def _combine_body(a_ref, b_ref, dd_ref, o_ref):
    s = a_ref[...] + b_ref[...]
    d = dd_ref[:, :1] + dd_ref[:, DEG_W:DEG_W + 1]
    o_ref[...] = s / jnp.maximum(d, 1e-8)


_SHIFT = N_NODES // _BLK

_combine = pl.pallas_call(
    _combine_body,
    grid=(N_NODES // _BLK,),
    in_specs=[
        pl.BlockSpec((_BLK, D_FEAT), lambda i: (i, 0)),
        pl.BlockSpec((_BLK, D_FEAT), lambda i: (i + _SHIFT, 0)),
        pl.BlockSpec((_BLK, D_FEAT), lambda i: (i, 0)),
    ],
    out_specs=pl.BlockSpec((_BLK, D_FEAT), lambda i: (i, 0)),
    out_shape=jax.ShapeDtypeStruct((N_NODES, D_FEAT), jnp.float32),
)


import numpy as _np

_ZEROS = _np.zeros((ROWS_PER_TILE, D_FEAT), _np.float32)
_ZEROD = _np.zeros((ROWS_PER_TILE, DEG_W), _np.float32)
_ONES = _np.ones((CHUNK, DEG_W), _np.float32)


@jax.jit
def kernel(x, edge_index):
    ei = jnp.transpose(
        edge_index.astype(jnp.int32).reshape(2, N_CHUNKS, CHUNK), (1, 0, 2))
    p, d = _sc_call(x, ei, _ZEROS, _ZEROD, _ONES)
    return _combine(p, p, d)


# speedup vs baseline: 1.0055x; 1.0055x over previous
import jax
import jax.numpy as jnp
from jax import lax
from jax.experimental import pallas as pl
from jax.experimental.pallas import tpu as pltpu
from jax.experimental.pallas import tpu_sc as plsc

N_NODES = 10000
N_EDGES = 320000
D_FEAT = 128
DEG_W = 8
NC, NS = 2, 16
NW = NC * NS
CHUNK = 128
N_CHUNKS = N_EDGES // CHUNK
CH_PER_TILE = N_CHUNKS // NW
N_TAIL = N_CHUNKS - CH_PER_TILE * NW
SEC_CHUNKS = 6
N_SEC = CH_PER_TILE // SEC_CHUNKS
ROWS_PER_TILE = N_NODES // NS


def _sc_body(x_hbm, ei_hbm, zeros_hbm, zerod_hbm, ones_hbm,
             out_hbm, deg_hbm,
             isec0, isec1, rows0_v, rows1_v, ones_v,
             acc_sh, deg_sh, isem0, isem1, gsem0, gsem1, osem):
    c = lax.axis_index("c")
    s = lax.axis_index("s")
    wid = c * NS + s
    base = wid * CH_PER_TILE

    isecs, isems = (isec0, isec1), (isem0, isem1)
    bufs, gsems = (rows0_v, rows1_v), (gsem0, gsem1)

    pltpu.async_copy(ei_hbm.at[pl.ds(base, SEC_CHUNKS)], isec0, isem0)
    pltpu.sync_copy(ones_hbm, ones_v)
    row0 = pl.multiple_of(s * ROWS_PER_TILE, 8)
    pltpu.sync_copy(zeros_hbm, acc_sh.at[pl.ds(row0, ROWS_PER_TILE)])
    pltpu.sync_copy(zerod_hbm, deg_sh.at[pl.ds(row0, ROWS_PER_TILE)])
    plsc.subcore_barrier()

    pltpu.make_async_copy(ei_hbm.at[pl.ds(base, SEC_CHUNKS)],
                          isec0, isem0).wait()
    pltpu.async_copy(x_hbm.at[isec0.at[0, 0]], rows0_v, gsem0)
    pltpu.async_copy(x_hbm.at[isec0.at[1, 0]], rows1_v, gsem1)

    for sec in range(N_SEC):
        ib = isecs[sec % 2]
        nb = (sec + 1) % 2
        if sec + 1 < N_SEC:
            pltpu.async_copy(
                ei_hbm.at[pl.ds(base + (sec + 1) * SEC_CHUNKS, SEC_CHUNKS)],
                isecs[nb], isems[nb])

        def body(j, carry):
            for b in (0, 1):
                k = 2 * j + b
                pltpu.make_async_copy(x_hbm.at[ib.at[k, 0]],
                                      bufs[b], gsems[b]).wait()
                pltpu.sync_copy(bufs[b], acc_sh.at[ib.at[k, 1]], add=True)
                pltpu.async_copy(x_hbm.at[ib.at[k + 2, 0]], bufs[b], gsems[b])
                pltpu.sync_copy(ones_v, deg_sh.at[ib.at[k, 1]], add=True)
            return carry

        lax.fori_loop(0, SEC_CHUNKS // 2 - 1, body, 0)

        for b in (0, 1):
            k = SEC_CHUNKS - 2 + b
            pltpu.make_async_copy(x_hbm.at[ib.at[k, 0]],
                                  bufs[b], gsems[b]).wait()
            pltpu.sync_copy(bufs[b], acc_sh.at[ib.at[k, 1]], add=True)
            if sec + 1 < N_SEC:
                if b == 0:
                    pltpu.make_async_copy(
                        ei_hbm.at[pl.ds(base + (sec + 1) * SEC_CHUNKS,
                                        SEC_CHUNKS)],
                        isecs[nb], isems[nb]).wait()
                pltpu.async_copy(x_hbm.at[isecs[nb].at[b, 0]],
                                 bufs[b], gsems[b])
            pltpu.sync_copy(ones_v, deg_sh.at[ib.at[k, 1]], add=True)

    pltpu.sync_copy(ei_hbm.at[pl.ds(NW * CH_PER_TILE, N_TAIL)],
                    isec0.at[pl.ds(0, N_TAIL)])
    cxl = wid // 8
    off = pl.multiple_of((wid % 8) * 16, 8)
    pltpu.async_copy(x_hbm.at[isec0.at[cxl, 0, pl.ds(off, 16)]],
                     rows0_v.at[pl.ds(0, 16)], gsem0).wait()
    pltpu.sync_copy(rows0_v.at[pl.ds(0, 16)],
                    acc_sh.at[isec0.at[cxl, 1, pl.ds(off, 16)]], add=True)
    pltpu.sync_copy(ones_v.at[pl.ds(0, 16)],
                    deg_sh.at[isec0.at[cxl, 1, pl.ds(off, 16)]], add=True)

    plsc.subcore_barrier()

    orow = pl.multiple_of(c * N_NODES + s * ROWS_PER_TILE, 8)
    pltpu.sync_copy(acc_sh.at[pl.ds(row0, ROWS_PER_TILE)],
                    out_hbm.at[pl.ds(orow, ROWS_PER_TILE)])
    ocol = pl.multiple_of(c * DEG_W, 8)
    pltpu.sync_copy(deg_sh.at[pl.ds(row0, ROWS_PER_TILE)],
                    deg_hbm.at[pl.ds(row0, ROWS_PER_TILE), pl.ds(ocol, DEG_W)])


_sc_call = pl.kernel(
    _sc_body,
    out_type=(
        jax.ShapeDtypeStruct((2 * N_NODES, D_FEAT), jnp.float32),
        jax.ShapeDtypeStruct((N_NODES, D_FEAT), jnp.float32),
    ),
    mesh=plsc.VectorSubcoreMesh(core_axis_name="c", subcore_axis_name="s"),
    compiler_params=pltpu.CompilerParams(use_tc_tiling_on_sc=False),
    scratch_types=(
        pltpu.VMEM((SEC_CHUNKS, 2, CHUNK), jnp.int32),
        pltpu.VMEM((SEC_CHUNKS, 2, CHUNK), jnp.int32),
        pltpu.VMEM((CHUNK, D_FEAT), jnp.float32),
        pltpu.VMEM((CHUNK, D_FEAT), jnp.float32),
        pltpu.VMEM((CHUNK, DEG_W), jnp.float32),
        pltpu.VMEM_SHARED((N_NODES, D_FEAT), jnp.float32),
        pltpu.VMEM_SHARED((N_NODES, DEG_W), jnp.float32),
        pltpu.SemaphoreType.DMA,
        pltpu.SemaphoreType.DMA,
        pltpu.SemaphoreType.DMA,
        pltpu.SemaphoreType.DMA,
        pltpu.SemaphoreType.DMA,
    ),
)


_BLK = 2000


def _combine_body(a_ref, b_ref, dd_ref, o_ref):
    s = a_ref[...] + b_ref[...]
    d = dd_ref[:, :1] + dd_ref[:, DEG_W:DEG_W + 1]
    o_ref[...] = s / jnp.maximum(d, 1e-8)


_SHIFT = N_NODES // _BLK

_combine = pl.pallas_call(
    _combine_body,
    grid=(N_NODES // _BLK,),
    in_specs=[
        pl.BlockSpec((_BLK, D_FEAT), lambda i: (i, 0)),
        pl.BlockSpec((_BLK, D_FEAT), lambda i: (i + _SHIFT, 0)),
        pl.BlockSpec((_BLK, D_FEAT), lambda i: (i, 0)),
    ],
    out_specs=pl.BlockSpec((_BLK, D_FEAT), lambda i: (i, 0)),
    out_shape=jax.ShapeDtypeStruct((N_NODES, D_FEAT), jnp.float32),
)


import numpy as _np

_ZEROS = _np.zeros((ROWS_PER_TILE, D_FEAT), _np.float32)
_ZEROD = _np.zeros((ROWS_PER_TILE, DEG_W), _np.float32)
_ONES = _np.ones((CHUNK, DEG_W), _np.float32)


@jax.jit
def kernel(x, edge_index):
    ei = jnp.transpose(
        edge_index.astype(jnp.int32).reshape(2, N_CHUNKS, CHUNK), (1, 0, 2))
    p, d = _sc_call(x, ei, _ZEROS, _ZEROD, _ONES)
    return _combine(p, p, d)
